# Initial kernel scaffold; baseline (speedup 1.0000x reference)
import jax
import jax.numpy as jnp
from jax.experimental import pallas as pl


def kernel(x, W, bW, A, B, sparse_values, sparse_indices):
    raise NotImplementedError("write your pallas kernel here")


# speedup vs baseline: 1.1633x; 1.1633x over previous
import functools

import jax
import jax.numpy as jnp
from jax import lax
from jax.experimental import pallas as pl
from jax.experimental.pallas import tpu as pltpu
from jax.experimental.pallas import tpu_sc as plsc

D_IN_C = 2048
D_OUT_C = 2048
RANK_C = 64

NUM_CORES = 2
NUM_SUBCORES = 16
NNZ_PER_SUB = 2688
NNZ_PAD = NNZ_PER_SUB * NUM_SUBCORES
GROUPS = NNZ_PER_SUB // 128

CHUNK_ROWS = 256
CHUNKS_PER_CORE = 1024 // CHUNK_ROWS
ROWS_PER_TILE = CHUNK_ROWS // NUM_SUBCORES
STAGE_ELEMS = ROWS_PER_TILE * D_IN_C


def _combine_body(w_ref, a_ref, b_ref, o_ref):
    ab = jax.lax.dot_general(
        a_ref[...], b_ref[...],
        dimension_numbers=(((1,), (0,)), ((), ())),
        preferred_element_type=jnp.float32)
    o_ref[...] = w_ref[...] + ab


def _combine(W, A16, B16):
    bm = 512
    return pl.pallas_call(
        _combine_body,
        grid=(D_OUT_C // bm,),
        in_specs=[
            pl.BlockSpec((bm, D_IN_C), lambda i: (i, 0)),
            pl.BlockSpec((bm, RANK_C), lambda i: (i, 0)),
            pl.BlockSpec((RANK_C, D_IN_C), lambda i: (0, 0)),
        ],
        out_specs=pl.BlockSpec((bm, D_IN_C), lambda i: (i, 0)),
        out_shape=jax.ShapeDtypeStruct((D_OUT_C, D_IN_C), jnp.float32),
    )(W, A16, B16)


def _scatter_body(m0_hbm, rows_hbm, cols_hbm, vals_hbm, m_out_hbm,
                  rows_v, cols_v, vals_v, idx2d, val2d, stage_v, shared):
    c = lax.axis_index("c")
    s = lax.axis_index("s")
    base = pl.multiple_of(s * NNZ_PER_SUB, NNZ_PER_SUB)
    pltpu.sync_copy(rows_hbm.at[pl.ds(base, NNZ_PER_SUB)], rows_v)
    pltpu.sync_copy(cols_hbm.at[pl.ds(base, NNZ_PER_SUB)], cols_v)
    pltpu.sync_copy(vals_hbm.at[pl.ds(base, NNZ_PER_SUB)], vals_v)

    def chunk_body(chunk, carry):
        lo = c * (CHUNKS_PER_CORE * CHUNK_ROWS) + chunk * CHUNK_ROWS
        g_base = pl.multiple_of((lo + s * ROWS_PER_TILE) * D_IN_C, D_IN_C)
        l_base = pl.multiple_of(s * STAGE_ELEMS, STAGE_ELEMS)
        pltpu.sync_copy(m0_hbm.at[pl.ds(g_base, STAGE_ELEMS)], stage_v)
        pltpu.sync_copy(stage_v, shared.at[pl.ds(l_base, STAGE_ELEMS)])
        plsc.subcore_barrier()
        for g in range(GROUPS):
            for k in range(128 // 16):
                i = g * 8 + k
                r = rows_v[pl.ds(i * 16, 16)]
                cc = cols_v[pl.ds(i * 16, 16)]
                v = vals_v[pl.ds(i * 16, 16)]
                inr = (r >= lo) & (r < lo + CHUNK_ROWS)
                idx = jnp.where(inr, (r - lo) * D_IN_C + cc, 0)
                vm = jnp.where(inr, v, jnp.float32(0.0))
                idx2d[g, pl.ds(k * 16, 16)] = idx
                val2d[g, pl.ds(k * 16, 16)] = vm
            pltpu.sync_copy(val2d.at[g], shared.at[idx2d.at[g]], add=True)
        plsc.subcore_barrier()
        pltpu.sync_copy(shared.at[pl.ds(l_base, STAGE_ELEMS)], stage_v)
        pltpu.sync_copy(stage_v, m_out_hbm.at[pl.ds(g_base, STAGE_ELEMS)])
        plsc.subcore_barrier()
        return carry

    lax.fori_loop(0, CHUNKS_PER_CORE, chunk_body, 0)


def _scatter_add(m0_flat, rows_p, cols_p, vals_p):
    mesh = plsc.VectorSubcoreMesh(core_axis_name="c", subcore_axis_name="s")
    fn = pl.kernel(
        _scatter_body,
        out_type=jax.ShapeDtypeStruct((D_OUT_C * D_IN_C,), jnp.float32),
        mesh=mesh,
        scratch_types=[
            pltpu.VMEM((NNZ_PER_SUB,), jnp.int32),
            pltpu.VMEM((NNZ_PER_SUB,), jnp.int32),
            pltpu.VMEM((NNZ_PER_SUB,), jnp.float32),
            pltpu.VMEM((GROUPS, 128), jnp.int32),
            pltpu.VMEM((GROUPS, 128), jnp.float32),
            pltpu.VMEM((STAGE_ELEMS,), jnp.float32),
            pltpu.VMEM_SHARED((CHUNK_ROWS * D_IN_C,), jnp.float32),
        ],
    )
    return fn(m0_flat, rows_p, cols_p, vals_p)


def _matmul_body(x_ref, m_ref, b_ref, o_ref):
    acc = jax.lax.dot_general(
        x_ref[...], m_ref[...],
        dimension_numbers=(((1,), (1,)), ((), ())),
        preferred_element_type=jnp.float32)
    o_ref[...] = acc + b_ref[...]


def _matmul(x2d, M, bW2d, bm=512, bn=1024):
    nt, _ = x2d.shape
    return pl.pallas_call(
        _matmul_body,
        grid=(nt // bm, D_OUT_C // bn),
        in_specs=[
            pl.BlockSpec((bm, D_IN_C), lambda i, j: (i, 0)),
            pl.BlockSpec((bn, D_IN_C), lambda i, j: (j, 0)),
            pl.BlockSpec((1, bn), lambda i, j: (0, j)),
        ],
        out_specs=pl.BlockSpec((bm, bn), lambda i, j: (i, j)),
        out_shape=jax.ShapeDtypeStruct((nt, D_OUT_C), jnp.float32),
    )(x2d, M, bW2d)


def kernel(x, W, bW, A, B, sparse_values, sparse_indices):
    A16 = A.astype(jnp.float16).astype(jnp.float32)
    B16 = B.astype(jnp.float16).astype(jnp.float32)
    vals = sparse_values.astype(jnp.float16).astype(jnp.float32)
    rows = sparse_indices[0].astype(jnp.int32)
    cols = sparse_indices[1].astype(jnp.int32)
    nnz = vals.shape[0]
    pad = NNZ_PAD - nnz
    rows_p = jnp.pad(rows, (0, pad))
    cols_p = jnp.pad(cols, (0, pad))
    vals_p = jnp.pad(vals, (0, pad))

    m0 = _combine(W, A16, B16)
    m = _scatter_add(m0.reshape(-1), rows_p, cols_p, vals_p)
    m = m.reshape(D_OUT_C, D_IN_C)

    b, sl, d = x.shape
    x2d = x.reshape(b * sl, d)
    out = _matmul(x2d, m, bW.reshape(1, D_OUT_C))
    return out.reshape(b, sl, D_OUT_C)
